# Initial kernel scaffold; baseline (speedup 1.0000x reference)
import jax
import jax.numpy as jnp
from jax.experimental import pallas as pl


def kernel(context_words, center_word, neg_words, in_embed, out_embed):
    raise NotImplementedError("write your pallas kernel here")


# speedup vs baseline: 4.6900x; 4.6900x over previous
import functools

import jax
import jax.numpy as jnp
from jax import lax
from jax.experimental import pallas as pl
from jax.experimental.pallas import tpu as pltpu
from jax.experimental.pallas import tpu_sc as plsc

NC, NS = 2, 16
NW = NC * NS
LANES = 16


def _sc_scores(ctx_flat, center, neg_flat, in_embed, out_embed, B, CTX, NEG, D):
    BW = B // NW
    C = 16
    NIT = BW // C
    NKC = D // LANES

    mesh = plsc.VectorSubcoreMesh(core_axis_name="c", subcore_axis_name="s")

    @functools.partial(
        pl.kernel,
        out_type=(
            jax.ShapeDtypeStruct((B,), jnp.float32),
            jax.ShapeDtypeStruct((B * NEG,), jnp.float32),
        ),
        mesh=mesh,
        compiler_params=pltpu.CompilerParams(
            needs_layout_passes=False, use_tc_tiling_on_sc=False),
        scratch_types=[
            pltpu.VMEM((BW * CTX,), jnp.int32),
            pltpu.VMEM((BW,), jnp.int32),
            pltpu.VMEM((BW * NEG,), jnp.int32),
            pltpu.VMEM((C * CTX, D), jnp.float32),
            pltpu.VMEM((C, D), jnp.float32),
            pltpu.VMEM((C * NEG, D), jnp.float32),
            pltpu.VMEM((C * LANES,), jnp.float32),
            pltpu.VMEM((C * NEG * LANES,), jnp.float32),
            pltpu.VMEM((BW,), jnp.float32),
            pltpu.VMEM((BW * NEG,), jnp.float32),
            pltpu.SemaphoreType.DMA,
        ],
    )
    def score_kernel(ctx_hbm, cen_hbm, neg_hbm, ine_hbm, oute_hbm,
                     pos_o_hbm, neg_o_hbm,
                     ctx_idx, cen_idx, neg_idx,
                     ctx_rows, pos_rows, neg_rows,
                     stage_pos, stage_neg, pos_buf, neg_buf, sem):
        wid = lax.axis_index("s") * NC + lax.axis_index("c")
        pltpu.sync_copy(ctx_hbm.at[pl.ds(wid * BW * CTX, BW * CTX)], ctx_idx)
        pltpu.sync_copy(cen_hbm.at[pl.ds(wid * BW, BW)], cen_idx)
        pltpu.sync_copy(neg_hbm.at[pl.ds(wid * BW * NEG, BW * NEG)], neg_idx)

        def colsum(stage, r0):
            base = lax.iota(jnp.int32, 16) * LANES + (r0 * LANES)
            acc = plsc.load_gather(stage, [base])
            for c in range(1, LANES):
                acc = acc + plsc.load_gather(stage, [base + c])
            return acc

        def body(i, carry):
            dmas = []
            nctx = C * CTX
            for h in range(2):
                dmas.append(pltpu.async_copy(
                    ine_hbm.at[ctx_idx.at[pl.ds(i * nctx + h * (nctx // 2),
                                                nctx // 2)]],
                    ctx_rows.at[pl.ds(h * (nctx // 2), nctx // 2)], sem))
            dmas.append(pltpu.async_copy(
                oute_hbm.at[cen_idx.at[pl.ds(i * C, C)]], pos_rows, sem))
            nneg = C * NEG
            for h in range(4):
                dmas.append(pltpu.async_copy(
                    oute_hbm.at[neg_idx.at[pl.ds(i * nneg + h * (nneg // 4),
                                                 nneg // 4)]],
                    neg_rows.at[pl.ds(h * (nneg // 4), nneg // 4)], sem))
            for d in dmas:
                d.wait()

            for b in range(C):
                m = [ctx_rows[b * CTX, pl.ds(k * LANES, LANES)]
                     for k in range(NKC)]
                for c in range(1, CTX):
                    row = b * CTX + c
                    m = [m[k] + ctx_rows[row, pl.ds(k * LANES, LANES)]
                         for k in range(NKC)]
                m = [mk * (1.0 / CTX) for mk in m]

                p = m[0] * pos_rows[b, pl.ds(0, LANES)]
                for k in range(1, NKC):
                    p = p + m[k] * pos_rows[b, pl.ds(k * LANES, LANES)]
                stage_pos[pl.ds(b * LANES, LANES)] = p

                for n in range(NEG):
                    row = b * NEG + n
                    q = m[0] * neg_rows[row, pl.ds(0, LANES)]
                    for k in range(1, NKC):
                        q = q + m[k] * neg_rows[row, pl.ds(k * LANES, LANES)]
                    stage_neg[pl.ds(row * LANES, LANES)] = q

            pos_buf[pl.ds(i * C, C)] = colsum(stage_pos, 0)
            for g in range(C * NEG // LANES):
                neg_buf[pl.ds(i * (C * NEG) + g * LANES, LANES)] = (
                    colsum(stage_neg, g * LANES))
            return carry

        lax.fori_loop(0, NIT, body, 0)
        pltpu.sync_copy(pos_buf, pos_o_hbm.at[pl.ds(wid * BW, BW)])
        pltpu.sync_copy(neg_buf, neg_o_hbm.at[pl.ds(wid * BW * NEG, BW * NEG)])

    return score_kernel(ctx_flat, center, neg_flat, in_embed, out_embed)


def _loss_from_scores(pos_score, neg_score_flat, B):
    pos2 = pos_score.reshape(-1, 128)
    neg2 = neg_score_flat.reshape(-1, 128)

    def body(p_ref, n_ref, o_ref):
        def neg_softplus(x):
            return jnp.minimum(x, 0.0) - jnp.log(1.0 + jnp.exp(-jnp.abs(x)))

        total = jnp.sum(neg_softplus(p_ref[...]))
        total = total + jnp.sum(neg_softplus(-n_ref[...]))
        o_ref[0, 0] = -total / B

    out = pl.pallas_call(
        body,
        out_shape=jax.ShapeDtypeStruct((1, 1), jnp.float32),
        out_specs=pl.BlockSpec(memory_space=pltpu.SMEM),
    )(pos2, neg2)
    return out[0, 0]


def kernel(context_words, center_word, neg_words, in_embed, out_embed):
    B, CTX = context_words.shape
    NEG = neg_words.shape[1]
    D = in_embed.shape[1]
    pos_score, neg_score = _sc_scores(
        context_words.reshape(-1), center_word, neg_words.reshape(-1),
        in_embed, out_embed, B, CTX, NEG, D)
    return _loss_from_scores(pos_score, neg_score, B)
